# Initial kernel scaffold; baseline (speedup 1.0000x reference)
import jax
import jax.numpy as jnp
from jax.experimental import pallas as pl


def kernel(pix2face, bary_coords, faces, attributes):
    raise NotImplementedError("write your pallas kernel here")


# speedup vs baseline: 41.1943x; 41.1943x over previous
import functools

import jax
import jax.numpy as jnp
from jax import lax
from jax.experimental import pallas as pl
from jax.experimental.pallas import tpu as pltpu
from jax.experimental.pallas import tpu_sc as plsc

_NC, _NS = 2, 16
_NW = _NC * _NS
_CH = 512
_GI = 128


def _render_sc(pix_flat, b0, b1, b2, faces_flat, attr_flat, *, P, HW, V, D):
    pw = P // _NW
    n_chunks = pw // _CH
    mesh = plsc.VectorSubcoreMesh(core_axis_name="c", subcore_axis_name="s",
                                  num_cores=_NC, num_subcores=_NS)

    @functools.partial(
        pl.kernel,
        out_type=jax.ShapeDtypeStruct((P, D), jnp.float32),
        mesh=mesh,
        compiler_params=pltpu.CompilerParams(
            needs_layout_passes=False, use_tc_tiling_on_sc=False),
        scratch_types=[
            pltpu.VMEM((_CH,), jnp.int32),
            pltpu.VMEM((_CH,), jnp.float32),
            pltpu.VMEM((_CH,), jnp.float32),
            pltpu.VMEM((_CH,), jnp.float32),
            pltpu.VMEM((3 * _CH,), jnp.int32),
            pltpu.VMEM((3 * _CH,), jnp.int32),
            pltpu.VMEM((3 * _CH, D), jnp.float32),
            pltpu.VMEM((_CH, D), jnp.float32),
            pltpu.SemaphoreType.DMA,
        ],
    )
    def k(pix_hbm, b0_hbm, b1_hbm, b2_hbm, faces_hbm, attr_hbm, out_hbm,
          pix_v, b0_v, b1_v, b2_v, fidx_v, vidx_v, attr_v, out_v, sem):
        wid = lax.axis_index("s") * _NC + lax.axis_index("c")
        lane = lax.iota(jnp.int32, 16)
        zero16 = lane * 0

        def chunk_body(ci, carry):
            base = wid * pw + ci * _CH
            b = base // HW
            bV = b * V

            pltpu.sync_copy(pix_hbm.at[pl.ds(base, _CH)], pix_v)
            pltpu.sync_copy(b0_hbm.at[pl.ds(base, _CH)], b0_v)
            pltpu.sync_copy(b1_hbm.at[pl.ds(base, _CH)], b1_v)
            pltpu.sync_copy(b2_hbm.at[pl.ds(base, _CH)], b2_v)

            def extract(j, c):
                t = pix_v[pl.ds(j * 16, 16)] * 3
                fidx_v[pl.ds(0 * _CH + j * 16, 16)] = t
                fidx_v[pl.ds(1 * _CH + j * 16, 16)] = t + 1
                fidx_v[pl.ds(2 * _CH + j * 16, 16)] = t + 2
                return c

            lax.fori_loop(0, _CH // 16, extract, 0, unroll=4)

            face_cps = [
                pltpu.async_copy(
                    faces_hbm.at[fidx_v.at[pl.ds(g * _GI, _GI)]],
                    vidx_v.at[pl.ds(g * _GI, _GI)],
                    sem,
                )
                for g in range(3 * _CH // _GI)
            ]
            for cp in face_cps:
                cp.wait()

            def addoff(j, c):
                v = vidx_v[pl.ds(j * 16, 16)]
                vidx_v[pl.ds(j * 16, 16)] = v + bV
                return c

            lax.fori_loop(0, 3 * _CH // 16, addoff, 0, unroll=4)

            attr_cps = [
                pltpu.async_copy(
                    attr_hbm.at[vidx_v.at[pl.ds(g * _GI, _GI)]],
                    attr_v.at[pl.ds(g * _GI, _GI)],
                    sem,
                )
                for g in range(3 * _CH // _GI)
            ]
            for cp in attr_cps:
                cp.wait()

            def interp(p, c):
                psplat = zero16 + p
                w0 = plsc.load_gather(b0_v, [psplat])
                w1 = plsc.load_gather(b1_v, [psplat])
                w2 = plsc.load_gather(b2_v, [psplat])
                for h in range(0, D, 16):
                    a0 = attr_v[p, pl.ds(h, 16)]
                    a1 = attr_v[_CH + p, pl.ds(h, 16)]
                    a2 = attr_v[2 * _CH + p, pl.ds(h, 16)]
                    out_v[p, pl.ds(h, 16)] = w0 * a0 + w1 * a1 + w2 * a2
                return c

            lax.fori_loop(0, _CH, interp, 0, unroll=4)

            pltpu.sync_copy(out_v, out_hbm.at[pl.ds(base, _CH)])
            return carry

        lax.fori_loop(0, n_chunks, chunk_body, 0)

    return k(pix_flat, b0, b1, b2, faces_flat, attr_flat)


def kernel(pix2face, bary_coords, faces, attributes):
    B, H, W = pix2face.shape
    Ba, V, D = attributes.shape
    F = faces.shape[0]
    P = B * H * W
    pix_flat = pix2face.reshape(P)
    b0 = bary_coords[..., 0].reshape(P)
    b1 = bary_coords[..., 1].reshape(P)
    b2 = bary_coords[..., 2].reshape(P)
    faces_flat = faces.reshape(3 * F)
    attr_flat = attributes.reshape(Ba * V, D)
    out = _render_sc(pix_flat, b0, b1, b2, faces_flat, attr_flat,
                     P=P, HW=H * W, V=V, D=D)
    out = out.reshape(B, H, W, D)
    mask = pix2face != -1
    return out, mask


# speedup vs baseline: 53.5299x; 1.2994x over previous
import functools

import jax
import jax.numpy as jnp
from jax import lax
from jax.experimental import pallas as pl
from jax.experimental.pallas import tpu as pltpu
from jax.experimental.pallas import tpu_sc as plsc

_NC, _NS = 2, 16
_NW = _NC * _NS
_CH = 512
_GI = 128


def _render_sc(pix_flat, b0, b1, b2, f0, f1, f2, attr_flat, *, P, HW, V, D):
    pw = P // _NW
    n_chunks = pw // _CH
    mesh = plsc.VectorSubcoreMesh(core_axis_name="c", subcore_axis_name="s",
                                  num_cores=_NC, num_subcores=_NS)

    @functools.partial(
        pl.kernel,
        out_type=jax.ShapeDtypeStruct((P, D), jnp.float32),
        mesh=mesh,
        compiler_params=pltpu.CompilerParams(
            needs_layout_passes=False, use_tc_tiling_on_sc=False),
        scratch_types=[
            pltpu.VMEM((_CH,), jnp.int32),
            pltpu.VMEM((_CH,), jnp.float32),
            pltpu.VMEM((_CH,), jnp.float32),
            pltpu.VMEM((_CH,), jnp.float32),
            pltpu.VMEM((3 * _CH,), jnp.int32),
            pltpu.VMEM((3 * _CH, D), jnp.float32),
            pltpu.VMEM((_CH, D), jnp.float32),
            pltpu.SemaphoreType.DMA,
        ],
    )
    def k(pix_hbm, b0_hbm, b1_hbm, b2_hbm, f0_hbm, f1_hbm, f2_hbm, attr_hbm,
          out_hbm, pix_v, b0_v, b1_v, b2_v, vidx_v, attr_v, out_v, sem):
        wid = lax.axis_index("s") * _NC + lax.axis_index("c")

        def chunk_body(ci, carry):
            base = wid * pw + ci * _CH
            b = base // HW
            bV = b * V

            pltpu.sync_copy(pix_hbm.at[pl.ds(base, _CH)], pix_v)
            pltpu.sync_copy(b0_hbm.at[pl.ds(base, _CH)], b0_v)
            pltpu.sync_copy(b1_hbm.at[pl.ds(base, _CH)], b1_v)
            pltpu.sync_copy(b2_hbm.at[pl.ds(base, _CH)], b2_v)

            face_cps = [
                pltpu.async_copy(
                    fk.at[pix_v.at[pl.ds(g * _GI, _GI)]],
                    vidx_v.at[pl.ds(kk * _CH + g * _GI, _GI)],
                    sem,
                )
                for kk, fk in enumerate((f0_hbm, f1_hbm, f2_hbm))
                for g in range(_CH // _GI)
            ]
            for cp in face_cps:
                cp.wait()

            attr_b = attr_hbm.at[pl.ds(bV, V)]
            attr_cps = [
                pltpu.async_copy(
                    attr_b.at[vidx_v.at[pl.ds(g * _GI, _GI)]],
                    attr_v.at[pl.ds(g * _GI, _GI)],
                    sem,
                )
                for g in range(3 * _CH // _GI)
            ]
            for cp in attr_cps:
                cp.wait()

            def interp(j, c):
                w0v = b0_v[pl.ds(j * 16, 16)]
                w1v = b1_v[pl.ds(j * 16, 16)]
                w2v = b2_v[pl.ds(j * 16, 16)]
                for t in range(16):
                    p = j * 16 + t
                    w0, w1, w2 = w0v[t], w1v[t], w2v[t]
                    for h in range(0, D, 16):
                        a0 = attr_v[p, pl.ds(h, 16)]
                        a1 = attr_v[_CH + p, pl.ds(h, 16)]
                        a2 = attr_v[2 * _CH + p, pl.ds(h, 16)]
                        out_v[p, pl.ds(h, 16)] = w0 * a0 + w1 * a1 + w2 * a2
                return c

            lax.fori_loop(0, _CH // 16, interp, 0)

            pltpu.sync_copy(out_v, out_hbm.at[pl.ds(base, _CH)])
            return carry

        lax.fori_loop(0, n_chunks, chunk_body, 0)

    return k(pix_flat, b0, b1, b2, f0, f1, f2, attr_flat)


def kernel(pix2face, bary_coords, faces, attributes):
    B, H, W = pix2face.shape
    Ba, V, D = attributes.shape
    P = B * H * W
    pix_flat = pix2face.reshape(P)
    b0 = bary_coords[..., 0].reshape(P)
    b1 = bary_coords[..., 1].reshape(P)
    b2 = bary_coords[..., 2].reshape(P)
    f0 = faces[:, 0]
    f1 = faces[:, 1]
    f2 = faces[:, 2]
    attr_flat = attributes.reshape(Ba * V, D)
    out = _render_sc(pix_flat, b0, b1, b2, f0, f1, f2, attr_flat,
                     P=P, HW=H * W, V=V, D=D)
    out = out.reshape(B, H, W, D)
    mask = pix2face != -1
    return out, mask


# speedup vs baseline: 58.1427x; 1.0862x over previous
import functools

import jax
import jax.numpy as jnp
from jax import lax
from jax.experimental import pallas as pl
from jax.experimental.pallas import tpu as pltpu
from jax.experimental.pallas import tpu_sc as plsc

_NC, _NS = 2, 16
_NW = _NC * _NS
_CH = 256
_GI = 128


def _render_sc(pix_flat, b0, b1, b2, f0, f1, f2, attr_flat, *, P, HW, V, D):
    pw = P // _NW
    n = pw // _CH
    mesh = plsc.VectorSubcoreMesh(core_axis_name="c", subcore_axis_name="s",
                                  num_cores=_NC, num_subcores=_NS)

    @functools.partial(
        pl.kernel,
        out_type=jax.ShapeDtypeStruct((P, D), jnp.float32),
        mesh=mesh,
        compiler_params=pltpu.CompilerParams(
            needs_layout_passes=False, use_tc_tiling_on_sc=False),
        scratch_types=[
            [pltpu.VMEM((_CH,), jnp.int32) for _ in range(2)],
            [pltpu.VMEM((_CH,), jnp.float32) for _ in range(4)],
            [pltpu.VMEM((_CH,), jnp.float32) for _ in range(4)],
            [pltpu.VMEM((_CH,), jnp.float32) for _ in range(4)],
            [pltpu.VMEM((3 * _CH,), jnp.int32) for _ in range(4)],
            [pltpu.VMEM((3 * _CH, D), jnp.float32) for _ in range(2)],
            [pltpu.VMEM((_CH, D), jnp.float32) for _ in range(2)],
            pltpu.SemaphoreType.DMA,
            [pltpu.SemaphoreType.DMA for _ in range(2)],
            [pltpu.SemaphoreType.DMA for _ in range(2)],
        ],
    )
    def k(pix_hbm, b0_hbm, b1_hbm, b2_hbm, f0_hbm, f1_hbm, f2_hbm, attr_hbm,
          out_hbm, pix_v, b0_v, b1_v, b2_v, vidx_v, attr_v, out_v,
          sem_f, sem_a, sem_o):
        wid = lax.axis_index("s") * _NC + lax.axis_index("c")

        def base_of(x):
            return wid * pw + x * _CH

        def sync_in(x, m4):
            base = base_of(x)
            pltpu.sync_copy(pix_hbm.at[pl.ds(base, _CH)], pix_v[m4 % 2])
            pltpu.sync_copy(b0_hbm.at[pl.ds(base, _CH)], b0_v[m4])
            pltpu.sync_copy(b1_hbm.at[pl.ds(base, _CH)], b1_v[m4])
            pltpu.sync_copy(b2_hbm.at[pl.ds(base, _CH)], b2_v[m4])

        def fire_faces(x, m4):
            for kk, fk in enumerate((f0_hbm, f1_hbm, f2_hbm)):
                for g in range(_CH // _GI):
                    pltpu.async_copy(
                        fk.at[pix_v[m4 % 2].at[pl.ds(g * _GI, _GI)]],
                        vidx_v[m4].at[pl.ds(kk * _CH + g * _GI, _GI)],
                        sem_f,
                    )

        def wait_faces(m4):
            pltpu.make_async_copy(
                f0_hbm.at[pl.ds(0, 3 * _CH)], vidx_v[m4], sem_f).wait()

        def fire_attr(x, m4):
            bV = (base_of(x) // HW) * V
            attr_b = attr_hbm.at[pl.ds(bV, V)]
            for g in range(3 * _CH // _GI):
                pltpu.async_copy(
                    attr_b.at[vidx_v[m4].at[pl.ds(g * _GI, _GI)]],
                    attr_v[m4 % 2].at[pl.ds(g * _GI, _GI)],
                    sem_a[m4 % 2],
                )

        def wait_attr(m4):
            pltpu.make_async_copy(
                attr_hbm.at[pl.ds(0, 3 * _CH)], attr_v[m4 % 2],
                sem_a[m4 % 2]).wait()

        def fire_out(x, m4):
            pltpu.async_copy(
                out_v[m4 % 2], out_hbm.at[pl.ds(base_of(x), _CH)],
                sem_o[m4 % 2])

        def wait_out(m4):
            pltpu.make_async_copy(
                out_v[m4 % 2], out_hbm.at[pl.ds(0, _CH)],
                sem_o[m4 % 2]).wait()

        def interp(x, m4):
            av, ov = attr_v[m4 % 2], out_v[m4 % 2]
            w0r, w1r, w2r = b0_v[m4], b1_v[m4], b2_v[m4]

            def body(j, c):
                w0v = w0r[pl.ds(j * 16, 16)]
                w1v = w1r[pl.ds(j * 16, 16)]
                w2v = w2r[pl.ds(j * 16, 16)]
                for t in range(16):
                    p = j * 16 + t
                    w0, w1, w2 = w0v[t], w1v[t], w2v[t]
                    for h in range(0, D, 16):
                        a0 = av[p, pl.ds(h, 16)]
                        a1 = av[_CH + p, pl.ds(h, 16)]
                        a2 = av[2 * _CH + p, pl.ds(h, 16)]
                        ov[p, pl.ds(h, 16)] = w0 * a0 + w1 * a1 + w2 * a2
                return c

            lax.fori_loop(0, _CH // 16, body, 0)

        def steady(i, m4, first, last):
            wait_faces(m4)
            fire_attr(i, m4)
            if last:
                @pl.when(i + 1 < n)
                def _stage_next():
                    sync_in(i + 1, (m4 + 1) % 4)
                    fire_faces(i + 1, (m4 + 1) % 4)
            else:
                sync_in(i + 1, (m4 + 1) % 4)
                fire_faces(i + 1, (m4 + 1) % 4)
            wait_attr((m4 - 1) % 4)
            if not first:
                wait_out((m4 - 1) % 4)
            interp(i - 1, (m4 - 1) % 4)
            fire_out(i - 1, (m4 - 1) % 4)

        sync_in(0, 0)
        fire_faces(0, 0)
        sync_in(1, 1)
        wait_faces(0)
        fire_attr(0, 0)
        fire_faces(1, 1)
        steady(1, 1, True, False)
        steady(2, 2, True, False)
        steady(3, 3, False, False)

        def block(blk, carry):
            i0 = 4 + blk * 4
            for ph in range(4):
                steady(i0 + ph, ph, False, ph == 3 and True)
            return carry

        lax.fori_loop(0, (n - 4) // 4, block, 0)

        wait_attr((n - 1) % 4)
        wait_out((n - 1) % 4)
        interp(n - 1, (n - 1) % 4)
        fire_out(n - 1, (n - 1) % 4)
        wait_out((n - 2) % 4)
        wait_out((n - 1) % 4)

    return k(pix_flat, b0, b1, b2, f0, f1, f2, attr_flat)


def kernel(pix2face, bary_coords, faces, attributes):
    B, H, W = pix2face.shape
    Ba, V, D = attributes.shape
    P = B * H * W
    pix_flat = pix2face.reshape(P)
    b0 = bary_coords[..., 0].reshape(P)
    b1 = bary_coords[..., 1].reshape(P)
    b2 = bary_coords[..., 2].reshape(P)
    f0 = faces[:, 0]
    f1 = faces[:, 1]
    f2 = faces[:, 2]
    attr_flat = attributes.reshape(Ba * V, D)
    out = _render_sc(pix_flat, b0, b1, b2, f0, f1, f2, attr_flat,
                     P=P, HW=H * W, V=V, D=D)
    out = out.reshape(B, H, W, D)
    mask = pix2face != -1
    return out, mask


# speedup vs baseline: 72.6138x; 1.2489x over previous
import functools

import jax
import jax.numpy as jnp
from jax import lax
from jax.experimental import pallas as pl
from jax.experimental.pallas import tpu as pltpu
from jax.experimental.pallas import tpu_sc as plsc

_NC, _NS = 2, 16
_NW = _NC * _NS
_CH = 256
_GI = 128


def _render_sc(pix_flat, b0, b1, b2, f0, f1, f2, attr_flat, *, P, HW, V, D):
    pw = P // _NW
    n = pw // _CH
    mesh = plsc.VectorSubcoreMesh(core_axis_name="c", subcore_axis_name="s",
                                  num_cores=_NC, num_subcores=_NS)

    @functools.partial(
        pl.kernel,
        out_type=jax.ShapeDtypeStruct((P, D), jnp.float32),
        mesh=mesh,
        compiler_params=pltpu.CompilerParams(
            needs_layout_passes=False, use_tc_tiling_on_sc=False),
        scratch_types=[
            [pltpu.VMEM((_CH,), jnp.int32) for _ in range(2)],
            [pltpu.VMEM((_CH,), jnp.float32) for _ in range(4)],
            [pltpu.VMEM((_CH,), jnp.float32) for _ in range(4)],
            [pltpu.VMEM((_CH,), jnp.float32) for _ in range(4)],
            [pltpu.VMEM((3 * _CH,), jnp.int32) for _ in range(4)],
            [pltpu.VMEM((3 * _CH, D), jnp.float32) for _ in range(2)],
            [pltpu.VMEM((_CH, D), jnp.float32) for _ in range(2)],
            pltpu.SemaphoreType.DMA,
            [pltpu.SemaphoreType.DMA for _ in range(2)],
            [pltpu.SemaphoreType.DMA for _ in range(2)],
            pltpu.SemaphoreType.DMA,
        ],
    )
    def k(pix_hbm, b0_hbm, b1_hbm, b2_hbm, f0_hbm, f1_hbm, f2_hbm, attr_hbm,
          out_hbm, pix_v, b0_v, b1_v, b2_v, vidx_v, attr_v, out_v,
          sem_f, sem_a, sem_o, sem_i):
        wid = lax.axis_index("s") * _NC + lax.axis_index("c")

        def base_of(x):
            return wid * pw + x * _CH

        def sync_in(x, m4):
            base = base_of(x)
            pltpu.sync_copy(pix_hbm.at[pl.ds(base, _CH)], pix_v[m4 % 2])
            pltpu.sync_copy(b0_hbm.at[pl.ds(base, _CH)], b0_v[m4])
            pltpu.sync_copy(b1_hbm.at[pl.ds(base, _CH)], b1_v[m4])
            pltpu.sync_copy(b2_hbm.at[pl.ds(base, _CH)], b2_v[m4])

        def fire_in(x, m4):
            base = base_of(x)
            pltpu.async_copy(pix_hbm.at[pl.ds(base, _CH)], pix_v[m4 % 2],
                             sem_i)
            pltpu.async_copy(b0_hbm.at[pl.ds(base, _CH)], b0_v[m4], sem_i)
            pltpu.async_copy(b1_hbm.at[pl.ds(base, _CH)], b1_v[m4], sem_i)
            pltpu.async_copy(b2_hbm.at[pl.ds(base, _CH)], b2_v[m4], sem_i)

        def wait_in(m4):
            pltpu.make_async_copy(
                pix_hbm.at[pl.ds(0, _CH)], pix_v[m4 % 2], sem_i).wait()
            pltpu.make_async_copy(
                b0_hbm.at[pl.ds(0, _CH)], b0_v[m4], sem_i).wait()
            pltpu.make_async_copy(
                b1_hbm.at[pl.ds(0, _CH)], b1_v[m4], sem_i).wait()
            pltpu.make_async_copy(
                b2_hbm.at[pl.ds(0, _CH)], b2_v[m4], sem_i).wait()

        def fire_faces(x, m4):
            for kk, fk in enumerate((f0_hbm, f1_hbm, f2_hbm)):
                for g in range(_CH // _GI):
                    pltpu.async_copy(
                        fk.at[pix_v[m4 % 2].at[pl.ds(g * _GI, _GI)]],
                        vidx_v[m4].at[pl.ds(kk * _CH + g * _GI, _GI)],
                        sem_f,
                    )

        def wait_faces(m4):
            pltpu.make_async_copy(
                f0_hbm.at[pl.ds(0, 3 * _CH)], vidx_v[m4], sem_f).wait()

        def fire_attr(x, m4):
            bV = (base_of(x) // HW) * V
            attr_b = attr_hbm.at[pl.ds(bV, V)]
            for g in range(3 * _CH // _GI):
                pltpu.async_copy(
                    attr_b.at[vidx_v[m4].at[pl.ds(g * _GI, _GI)]],
                    attr_v[m4 % 2].at[pl.ds(g * _GI, _GI)],
                    sem_a[m4 % 2],
                )

        def wait_attr(m4):
            pltpu.make_async_copy(
                attr_hbm.at[pl.ds(0, 3 * _CH)], attr_v[m4 % 2],
                sem_a[m4 % 2]).wait()

        def fire_out(x, m4):
            pltpu.async_copy(
                out_v[m4 % 2], out_hbm.at[pl.ds(base_of(x), _CH)],
                sem_o[m4 % 2])

        def wait_out(m4):
            pltpu.make_async_copy(
                out_v[m4 % 2], out_hbm.at[pl.ds(0, _CH)],
                sem_o[m4 % 2]).wait()

        def interp(x, m4):
            av, ov = attr_v[m4 % 2], out_v[m4 % 2]
            w0r, w1r, w2r = b0_v[m4], b1_v[m4], b2_v[m4]

            def body(j, c):
                w0v = w0r[pl.ds(j * 16, 16)]
                w1v = w1r[pl.ds(j * 16, 16)]
                w2v = w2r[pl.ds(j * 16, 16)]
                for t in range(16):
                    p = j * 16 + t
                    w0, w1, w2 = w0v[t], w1v[t], w2v[t]
                    for h in range(0, D, 16):
                        a0 = av[p, pl.ds(h, 16)]
                        a1 = av[_CH + p, pl.ds(h, 16)]
                        a2 = av[2 * _CH + p, pl.ds(h, 16)]
                        ov[p, pl.ds(h, 16)] = w0 * a0 + w1 * a1 + w2 * a2
                return c

            lax.fori_loop(0, _CH // 16, body, 0)

        def steady(i, m4, first, guard1, guard2):
            wait_faces(m4)
            fire_attr(i, m4)
            if guard1:
                @pl.when(i + 1 < n)
                def _stage_next():
                    wait_in((m4 + 1) % 4)
                    fire_faces(i + 1, (m4 + 1) % 4)
            else:
                wait_in((m4 + 1) % 4)
                fire_faces(i + 1, (m4 + 1) % 4)
            wait_attr((m4 - 1) % 4)
            if not first:
                wait_out((m4 - 1) % 4)
            interp(i - 1, (m4 - 1) % 4)
            fire_out(i - 1, (m4 - 1) % 4)
            if guard2:
                @pl.when(i + 2 < n)
                def _prefetch_in():
                    fire_in(i + 2, (m4 + 2) % 4)
            else:
                fire_in(i + 2, (m4 + 2) % 4)

        sync_in(0, 0)
        fire_faces(0, 0)
        sync_in(1, 1)
        wait_faces(0)
        fire_attr(0, 0)
        fire_faces(1, 1)
        fire_in(2, 2)
        steady(1, 1, True, False, False)
        steady(2, 2, True, False, False)
        steady(3, 3, False, False, False)

        def block(blk, carry):
            i0 = 4 + blk * 4
            for ph in range(4):
                steady(i0 + ph, ph, False, ph == 3, ph >= 2)
            return carry

        lax.fori_loop(0, (n - 4) // 4, block, 0)

        wait_attr((n - 1) % 4)
        wait_out((n - 1) % 4)
        interp(n - 1, (n - 1) % 4)
        fire_out(n - 1, (n - 1) % 4)
        wait_out((n - 2) % 4)
        wait_out((n - 1) % 4)

    return k(pix_flat, b0, b1, b2, f0, f1, f2, attr_flat)


def kernel(pix2face, bary_coords, faces, attributes):
    B, H, W = pix2face.shape
    Ba, V, D = attributes.shape
    P = B * H * W
    pix_flat = pix2face.reshape(P)
    b0 = bary_coords[..., 0].reshape(P)
    b1 = bary_coords[..., 1].reshape(P)
    b2 = bary_coords[..., 2].reshape(P)
    f0 = faces[:, 0]
    f1 = faces[:, 1]
    f2 = faces[:, 2]
    attr_flat = attributes.reshape(Ba * V, D)
    out = _render_sc(pix_flat, b0, b1, b2, f0, f1, f2, attr_flat,
                     P=P, HW=H * W, V=V, D=D)
    out = out.reshape(B, H, W, D)
    mask = pix2face != -1
    return out, mask
